# Initial kernel scaffold; baseline (speedup 1.0000x reference)
import jax
import jax.numpy as jnp
from jax.experimental import pallas as pl


def kernel(x_obstacle, x_agent, x_goal, ei_ona, ei_ana, ei_tow, ea_ona, ea_ana, ea_tow, action, params):
    raise NotImplementedError("write your pallas kernel here")


# speedup vs baseline: 1.1816x; 1.1816x over previous
import functools

import jax
import jax.numpy as jnp
from jax.experimental import pallas as pl
from jax.experimental.pallas import tpu as pltpu

H = 128
BE = 2000


def _b2(b):
    return b.reshape(1, -1)


def _embed_fx_body(ea_raw_ref, w1, b1, w2, b2, f1, c1, f2, c2,
                   ea_out, vals_out):
    x = ea_raw_ref[...]
    t = jnp.maximum(jnp.dot(x, w1[...], preferred_element_type=jnp.float32)
                    + b1[...], 0.0)
    ea = jnp.dot(t, w2[...], preferred_element_type=jnp.float32) + b2[...]
    u = jnp.maximum(jnp.dot(ea, f1[...], preferred_element_type=jnp.float32)
                    + c1[...], 0.0)
    vals = jnp.dot(u, f2[...], preferred_element_type=jnp.float32) + c2[...]
    if ea_out is not None:
        ea_out[...] = ea
    vals_out[...] = vals


def _embed_fx(ea_raw, emb, fx, want_ea):
    E = ea_raw.shape[0]
    grid = (E // BE,)
    w_spec = pl.BlockSpec((H, H), lambda i: (0, 0))
    b_spec = pl.BlockSpec((1, H), lambda i: (0, 0))
    in_specs = [
        pl.BlockSpec((BE, 16), lambda i: (i, 0)),
        pl.BlockSpec((16, H), lambda i: (0, 0)), b_spec, w_spec, b_spec,
        w_spec, b_spec, w_spec, b_spec,
    ]
    out_spec = pl.BlockSpec((BE, H), lambda i: (i, 0))
    if want_ea:
        out_shape = (jax.ShapeDtypeStruct((E, H), jnp.float32),
                     jax.ShapeDtypeStruct((E, H), jnp.float32))
        body = functools.partial(_embed_fx_body)
        fn = pl.pallas_call(
            lambda *refs: body(*refs[:9], refs[9], refs[10]),
            grid=grid, in_specs=in_specs,
            out_specs=(out_spec, out_spec), out_shape=out_shape)
    else:
        out_shape = jax.ShapeDtypeStruct((E, H), jnp.float32)
        fn = pl.pallas_call(
            lambda *refs: _embed_fx_body(*refs[:9], None, refs[9]),
            grid=grid, in_specs=in_specs,
            out_specs=out_spec, out_shape=out_shape)
    return fn(ea_raw, emb["W1"], _b2(emb["b1"]), emb["W2"], _b2(emb["b2"]),
              fx["W1"], _b2(fx["b1"]), fx["W2"], _b2(fx["b2"]))


def _update_fx_body(ea_ref, hg_ref, m1a, m1b, bm1, m2, bm2, f1, c1, f2, c2,
                    ea_out, vals_out):
    ea = ea_ref[...]
    hg = hg_ref[...]
    t = jnp.maximum(
        jnp.dot(ea, m1a[...], preferred_element_type=jnp.float32)
        + jnp.dot(hg, m1b[...], preferred_element_type=jnp.float32)
        + bm1[...], 0.0)
    ea2 = ea + jnp.dot(t, m2[...], preferred_element_type=jnp.float32) + bm2[...]
    u = jnp.maximum(jnp.dot(ea2, f1[...], preferred_element_type=jnp.float32)
                    + c1[...], 0.0)
    vals = jnp.dot(u, f2[...], preferred_element_type=jnp.float32) + c2[...]
    if ea_out is not None:
        ea_out[...] = ea2
    vals_out[...] = vals


def _update_fx(ea, hg, mlp, fx, want_ea):
    E = ea.shape[0]
    grid = (E // BE,)
    w_spec = pl.BlockSpec((H, H), lambda i: (0, 0))
    b_spec = pl.BlockSpec((1, H), lambda i: (0, 0))
    e_spec = pl.BlockSpec((BE, H), lambda i: (i, 0))
    in_specs = [e_spec, e_spec,
                w_spec, w_spec, b_spec, w_spec, b_spec,
                w_spec, b_spec, w_spec, b_spec]
    m1a = mlp["W1"][:H]
    m1b = mlp["W1"][H:]
    if want_ea:
        out_shape = (jax.ShapeDtypeStruct((E, H), jnp.float32),
                     jax.ShapeDtypeStruct((E, H), jnp.float32))
        fn = pl.pallas_call(
            lambda *refs: _update_fx_body(*refs[:11], refs[11], refs[12]),
            grid=grid, in_specs=in_specs,
            out_specs=(e_spec, e_spec), out_shape=out_shape)
    else:
        out_shape = jax.ShapeDtypeStruct((E, H), jnp.float32)
        fn = pl.pallas_call(
            lambda *refs: _update_fx_body(*refs[:11], None, refs[11]),
            grid=grid, in_specs=in_specs,
            out_specs=e_spec, out_shape=out_shape)
    return fn(ea, hg, m1a, m1b, _b2(mlp["b1"]), mlp["W2"], _b2(mlp["b2"]),
              fx["W1"], _b2(fx["b1"]), fx["W2"], _b2(fx["b2"]))


def _field_body(vec_ref, act_ref, w1v, w1a, b1, w2, b2, out_ref):
    t = jnp.maximum(
        jnp.dot(vec_ref[...], w1v[...], preferred_element_type=jnp.float32)
        + jnp.dot(act_ref[...], w1a[...], preferred_element_type=jnp.float32)
        + b1[...], 0.0)
    out_ref[...] = jnp.dot(t, w2[...], preferred_element_type=jnp.float32) + b2[...]


def _field(vec, action, p):
    n = vec.shape[0]
    w1v = p["W1"][:H]
    w1a = p["W1"][H:]
    fn = pl.pallas_call(
        _field_body,
        out_shape=jax.ShapeDtypeStruct((n, 1), jnp.float32))
    return fn(vec, action, w1v, w1a, _b2(p["b1"]), p["W2"],
              _b2(p["b2"])).squeeze(-1)


def _segmax(vals, dst, n):
    agg = jax.ops.segment_max(vals, dst, num_segments=n)
    return jnp.where(jnp.isneginf(agg), 0.0, agg)


def _process_type(ea_raw, dst, p, n):
    n_layers = len(p["layers"])
    want_ea = n_layers > 1
    res = _embed_fx(ea_raw, p["embed"], p["layers"][0]["fx"], want_ea)
    if want_ea:
        ea, vals = res
    else:
        vals = res
    h = _segmax(vals, dst, n)
    for li in range(1, n_layers):
        hg = h[dst]
        last = li == n_layers - 1
        res = _update_fx(ea, hg, p["layers"][li - 1]["mlp"],
                         p["layers"][li]["fx"], not last)
        if not last:
            ea, vals = res
        else:
            vals = res
        h = h + _segmax(vals, dst, n)
    return h


def kernel(x_obstacle, x_agent, x_goal, ei_ona, ei_ana, ei_tow,
           ea_ona, ea_ana, ea_tow, action, params):
    n = x_agent.shape[0]
    h_ona = _process_type(ea_ona, ei_ona[1], params["ona"], n)
    h_ana = _process_type(ea_ana, ei_ana[1], params["ana"], n)
    h_tow = _process_type(ea_tow, ei_tow[1], params["tow"], n)
    vec = jnp.maximum(jnp.maximum(jnp.maximum(h_ona, h_ana), h_tow), 0.0)
    return _field(vec, action, params["field"])
